# baseline (speedup 1.0000x reference)
import functools

import jax
import jax.numpy as jnp
import numpy as np
from jax import lax
from jax.experimental import pallas as pl
from jax.experimental.pallas import tpu as pltpu
from jax.experimental.pallas import tpu_sc as plsc

D_IN = 896
D_OUT = 768
MAX_LEN = 128

NC = 2
NS = 16
NW = NC * NS
GR = 32
NJ = D_IN // 32

_PERM = np.arange(D_IN).reshape(NJ, 2, 16).transpose(0, 2, 1).reshape(-1)


def _sc_gather(ids_3d, emb_i32):
    _, n_g, _ = ids_3d.shape
    n = ids_3d.size
    b_per_w = n // NW

    mesh = plsc.VectorSubcoreMesh(core_axis_name="c", subcore_axis_name="s")

    @functools.partial(
        pl.kernel,
        out_type=jax.ShapeDtypeStruct((n, D_IN // 2), jnp.int32),
        mesh=mesh,
        scratch_types=[
            pltpu.VMEM((n_g, GR), jnp.int32),
            pltpu.VMEM((GR, D_IN), jnp.int32),
            pltpu.VMEM((GR, D_IN), jnp.int32),
            pltpu.VMEM((GR, D_IN // 2), jnp.int32),
            pltpu.VMEM((GR, D_IN // 2), jnp.int32),
            pltpu.SemaphoreType.DMA,
            pltpu.SemaphoreType.DMA,
            pltpu.SemaphoreType.DMA,
            pltpu.SemaphoreType.DMA,
        ],
    )
    def k(ids_hbm, emb_hbm, out_hbm, idx_v, f0, f1, b0, b1, sg0, sg1, sw0, sw1):
        wid = lax.axis_index("s") * NC + lax.axis_index("c")
        base = wid * b_per_w
        pltpu.sync_copy(ids_hbm.at[wid], idx_v)

        mask_hi = jnp.int32(-65536)
        rnd = jnp.int32(0x8000)

        def convert(fbuf, bbuf):
            @functools.partial(plsc.parallel_loop, 0, GR, unroll=8)
            def _row(r):
                for j in range(NJ):
                    u = fbuf[r, pl.ds(32 * j, 16)]
                    v = fbuf[r, pl.ds(32 * j + 16, 16)]
                    lo = lax.shift_right_logical(u + rnd, 16)
                    hi = (v + rnd) & mask_hi
                    bbuf[r, pl.ds(16 * j, 16)] = hi | lo

        pltpu.async_copy(emb_hbm.at[idx_v.at[0]], f0, sg0)

        def body(p, carry):
            g0 = 2 * p
            pltpu.async_copy(emb_hbm.at[idx_v.at[g0 + 1]], f1, sg1)
            pltpu.make_async_copy(emb_hbm.at[idx_v.at[g0]], f0, sg0).wait()

            @pl.when(p > 0)
            def _():
                pltpu.make_async_copy(
                    b0, out_hbm.at[pl.ds(base + (g0 - 2) * GR, GR)], sw0
                ).wait()

            convert(f0, b0)
            pltpu.async_copy(b0, out_hbm.at[pl.ds(base + g0 * GR, GR)], sw0)

            @pl.when(g0 + 2 < n_g)
            def _():
                pltpu.async_copy(emb_hbm.at[idx_v.at[g0 + 2]], f0, sg0)

            pltpu.make_async_copy(emb_hbm.at[idx_v.at[g0 + 1]], f1, sg1).wait()

            @pl.when(p > 0)
            def _():
                pltpu.make_async_copy(
                    b1, out_hbm.at[pl.ds(base + (g0 - 1) * GR, GR)], sw1
                ).wait()

            convert(f1, b1)
            pltpu.async_copy(b1, out_hbm.at[pl.ds(base + (g0 + 1) * GR, GR)], sw1)
            return carry

        lax.fori_loop(0, n_g // 2, body, 0)

        pltpu.make_async_copy(
            b0, out_hbm.at[pl.ds(base + (n_g - 2) * GR, GR)], sw0
        ).wait()
        pltpu.make_async_copy(
            b1, out_hbm.at[pl.ds(base + (n_g - 1) * GR, GR)], sw1
        ).wait()

    return k(ids_3d, emb_i32)


def _mm_body(x_ref, w_ref, pos_ref, o_ref):
    o_ref[...] = (
        jnp.dot(x_ref[...], w_ref[...], preferred_element_type=jnp.float32)
        + pos_ref[...]
    )


def _tc_project_chunk(y, x_c, w_bf, pos_tiled, n_total, c0, block_m):
    steps = x_c.shape[0] // block_m
    out_spec = pl.BlockSpec((block_m, D_OUT), lambda i, c0=c0: (c0 + i, 0))
    in_specs = [
        pl.BlockSpec((block_m, D_IN), lambda i: (i, 0)),
        pl.BlockSpec((D_IN, D_OUT), lambda i: (0, 0)),
        pl.BlockSpec((block_m, D_OUT), lambda i: (0, 0)),
    ]
    out_shape = jax.ShapeDtypeStruct((n_total, D_OUT), jnp.float32)
    if y is None:
        return pl.pallas_call(
            _mm_body,
            grid=(steps,),
            in_specs=in_specs,
            out_specs=out_spec,
            out_shape=out_shape,
        )(x_c, w_bf, pos_tiled)

    def mm_alias(y_ref, x_ref, w_ref, pos_ref, o_ref):
        _mm_body(x_ref, w_ref, pos_ref, o_ref)

    return pl.pallas_call(
        mm_alias,
        grid=(steps,),
        in_specs=[pl.BlockSpec(memory_space=pl.ANY)] + in_specs,
        out_specs=out_spec,
        out_shape=out_shape,
        input_output_aliases={0: 0},
    )(y, x_c, w_bf, pos_tiled)


def kernel(input_ids, emb_weight, proj_weight, pos_weight):
    b, l = input_ids.shape
    n = b * l
    n_chunks = 4
    block_m = 1024
    nc = n // n_chunks

    ids_flat = input_ids.reshape(-1).astype(jnp.int32)
    w_bf = proj_weight.T[_PERM, :].astype(jnp.bfloat16)
    emb_i32 = lax.bitcast_convert_type(emb_weight, jnp.int32)
    pos_tiled = jnp.tile(pos_weight, (block_m // l, 1))

    gathered = [
        lax.bitcast_convert_type(
            _sc_gather(
                lax.dynamic_slice_in_dim(ids_flat, c * nc, nc).reshape(
                    NW, nc // (NW * GR), GR
                ),
                emb_i32,
            ),
            jnp.bfloat16,
        ).reshape(nc, D_IN)
        for c in range(n_chunks)
    ]
    y = None
    for c in range(n_chunks):
        y = _tc_project_chunk(
            y, gathered[c], w_bf, pos_tiled, n, c * (nc // block_m), block_m
        )
    return y.reshape(b, l, D_OUT)


# speedup vs baseline: 4.0219x; 4.0219x over previous
import functools

import jax
import jax.numpy as jnp
from jax import lax
from jax.experimental import pallas as pl
from jax.experimental.pallas import tpu as pltpu
from jax.experimental.pallas import tpu_sc as plsc

D_IN = 896
D_OUT = 768
MAX_LEN = 128

NC = 2
NS = 16
NW = NC * NS
GR = 64


def _sc_gather(ids_3d, emb_weight):
    _, n_g, _ = ids_3d.shape
    n = ids_3d.size
    b_per_w = n // NW

    mesh = plsc.VectorSubcoreMesh(core_axis_name="c", subcore_axis_name="s")

    @functools.partial(
        pl.kernel,
        out_type=jax.ShapeDtypeStruct((n, D_IN), jnp.float32),
        mesh=mesh,
        scratch_types=[
            pltpu.VMEM((n_g, GR), jnp.int32),
            pltpu.VMEM((GR, D_IN), jnp.float32),
            pltpu.VMEM((GR, D_IN), jnp.float32),
            pltpu.SemaphoreType.DMA,
            pltpu.SemaphoreType.DMA,
        ],
    )
    def k(ids_hbm, emb_hbm, out_hbm, idx_v, buf0, buf1, sem0, sem1):
        wid = lax.axis_index("s") * NC + lax.axis_index("c")
        base = wid * b_per_w
        pltpu.sync_copy(ids_hbm.at[wid], idx_v)

        pltpu.async_copy(emb_hbm.at[idx_v.at[0]], buf0, sem0)

        def body(p, carry):
            g0 = 2 * p
            pltpu.async_copy(emb_hbm.at[idx_v.at[g0 + 1]], buf1, sem1)
            pltpu.make_async_copy(emb_hbm.at[idx_v.at[g0]], buf0, sem0).wait()
            pltpu.sync_copy(buf0, out_hbm.at[pl.ds(base + g0 * GR, GR)])

            @pl.when(g0 + 2 < n_g)
            def _():
                pltpu.async_copy(emb_hbm.at[idx_v.at[g0 + 2]], buf0, sem0)

            pltpu.make_async_copy(emb_hbm.at[idx_v.at[g0 + 1]], buf1, sem1).wait()
            pltpu.sync_copy(buf1, out_hbm.at[pl.ds(base + (g0 + 1) * GR, GR)])
            return carry

        lax.fori_loop(0, n_g // 2, body, 0)

    return k(ids_3d, emb_weight)


def _mm_body(x_ref, w_ref, pos_ref, o_ref):
    xb = x_ref[...].astype(jnp.bfloat16)
    o_ref[...] = (
        jnp.dot(xb, w_ref[...], preferred_element_type=jnp.float32) + pos_ref[...]
    )


def _tc_project_chunk(y, x_c, w_bf, pos_tiled, n_total, c0, block_m):
    steps = x_c.shape[0] // block_m
    out_spec = pl.BlockSpec((block_m, D_OUT), lambda i, c0=c0: (c0 + i, 0))
    in_specs = [
        pl.BlockSpec((block_m, D_IN), lambda i: (i, 0)),
        pl.BlockSpec((D_IN, D_OUT), lambda i: (0, 0)),
        pl.BlockSpec((block_m, D_OUT), lambda i: (0, 0)),
    ]
    out_shape = jax.ShapeDtypeStruct((n_total, D_OUT), jnp.float32)
    if y is None:
        return pl.pallas_call(
            _mm_body,
            grid=(steps,),
            in_specs=in_specs,
            out_specs=out_spec,
            out_shape=out_shape,
        )(x_c, w_bf, pos_tiled)

    def mm_alias(y_ref, x_ref, w_ref, pos_ref, o_ref):
        _mm_body(x_ref, w_ref, pos_ref, o_ref)

    return pl.pallas_call(
        mm_alias,
        grid=(steps,),
        in_specs=[pl.BlockSpec(memory_space=pl.ANY)] + in_specs,
        out_specs=out_spec,
        out_shape=out_shape,
        input_output_aliases={0: 0},
    )(y, x_c, w_bf, pos_tiled)


def kernel(input_ids, emb_weight, proj_weight, pos_weight):
    b, l = input_ids.shape
    n = b * l
    n_chunks = 4
    block_m = 2048
    nc = n // n_chunks

    ids_flat = input_ids.reshape(-1).astype(jnp.int32)
    w_bf = proj_weight.T.astype(jnp.bfloat16)
    pos_tiled = jnp.tile(pos_weight, (block_m // l, 1))

    gathered = [
        _sc_gather(
            lax.dynamic_slice_in_dim(ids_flat, c * nc, nc).reshape(
                NW, nc // (NW * GR), GR
            ),
            emb_weight,
        )
        for c in range(n_chunks)
    ]
    y = None
    for c in range(n_chunks):
        y = _tc_project_chunk(
            y, gathered[c], w_bf, pos_tiled, n, c * (nc // block_m), block_m
        )
    return y.reshape(b, l, D_OUT)


# speedup vs baseline: 4.0427x; 1.0052x over previous
import functools

import jax
import jax.numpy as jnp
from jax import lax
from jax.experimental import pallas as pl
from jax.experimental.pallas import tpu as pltpu
from jax.experimental.pallas import tpu_sc as plsc

D_IN = 896
D_OUT = 768
MAX_LEN = 128

NC = 2
NS = 16
NW = NC * NS
GR = 64


def _sc_gather(ids_3d, emb_weight):
    _, n_g, _ = ids_3d.shape
    n = ids_3d.size
    b_per_w = n // NW

    mesh = plsc.VectorSubcoreMesh(core_axis_name="c", subcore_axis_name="s")

    @functools.partial(
        pl.kernel,
        out_type=jax.ShapeDtypeStruct((n, D_IN), jnp.float32),
        mesh=mesh,
        scratch_types=[
            pltpu.VMEM((n_g, GR), jnp.int32),
            pltpu.VMEM((GR, D_IN), jnp.float32),
            pltpu.VMEM((GR, D_IN), jnp.float32),
            pltpu.SemaphoreType.DMA,
            pltpu.SemaphoreType.DMA,
        ],
    )
    def k(ids_hbm, emb_hbm, out_hbm, idx_v, buf0, buf1, sem0, sem1):
        wid = lax.axis_index("s") * NC + lax.axis_index("c")
        base = wid * b_per_w
        pltpu.sync_copy(ids_hbm.at[wid], idx_v)

        pltpu.async_copy(emb_hbm.at[idx_v.at[0]], buf0, sem0)

        def body(p, carry):
            g0 = 2 * p
            pltpu.async_copy(emb_hbm.at[idx_v.at[g0 + 1]], buf1, sem1)
            pltpu.make_async_copy(emb_hbm.at[idx_v.at[g0]], buf0, sem0).wait()
            pltpu.sync_copy(buf0, out_hbm.at[pl.ds(base + g0 * GR, GR)])

            @pl.when(g0 + 2 < n_g)
            def _():
                pltpu.async_copy(emb_hbm.at[idx_v.at[g0 + 2]], buf0, sem0)

            pltpu.make_async_copy(emb_hbm.at[idx_v.at[g0 + 1]], buf1, sem1).wait()
            pltpu.sync_copy(buf1, out_hbm.at[pl.ds(base + (g0 + 1) * GR, GR)])
            return carry

        lax.fori_loop(0, n_g // 2, body, 0)

    return k(ids_3d, emb_weight)


def _mm_body(x_ref, w_ref, pos_ref, o_ref):
    xb = x_ref[...].astype(jnp.bfloat16)
    o_ref[...] = (
        jnp.dot(xb, w_ref[...], preferred_element_type=jnp.float32) + pos_ref[...]
    )


def _tc_project_chunk(y, x_c, w_bf, pos_tiled, n_total, c0, block_m):
    steps = x_c.shape[0] // block_m
    out_spec = pl.BlockSpec((block_m, D_OUT), lambda i, c0=c0: (c0 + i, 0))
    in_specs = [
        pl.BlockSpec((block_m, D_IN), lambda i: (i, 0)),
        pl.BlockSpec((D_IN, D_OUT), lambda i: (0, 0)),
        pl.BlockSpec((block_m, D_OUT), lambda i: (0, 0)),
    ]
    out_shape = jax.ShapeDtypeStruct((n_total, D_OUT), jnp.float32)
    if y is None:
        return pl.pallas_call(
            _mm_body,
            grid=(steps,),
            in_specs=in_specs,
            out_specs=out_spec,
            out_shape=out_shape,
        )(x_c, w_bf, pos_tiled)

    def mm_alias(y_ref, x_ref, w_ref, pos_ref, o_ref):
        _mm_body(x_ref, w_ref, pos_ref, o_ref)

    return pl.pallas_call(
        mm_alias,
        grid=(steps,),
        in_specs=[pl.BlockSpec(memory_space=pl.ANY)] + in_specs,
        out_specs=out_spec,
        out_shape=out_shape,
        input_output_aliases={0: 0},
    )(y, x_c, w_bf, pos_tiled)


def kernel(input_ids, emb_weight, proj_weight, pos_weight):
    b, l = input_ids.shape
    n = b * l
    block_m = 2048
    chunk_sizes = [16384, 38912, 38912, 36864]

    ids_flat = input_ids.reshape(-1).astype(jnp.int32)
    w_bf = proj_weight.T.astype(jnp.bfloat16)
    pos_tiled = jnp.tile(pos_weight, (block_m // l, 1))

    offs = [0]
    for s in chunk_sizes:
        offs.append(offs[-1] + s)
    gathered = [
        _sc_gather(
            lax.dynamic_slice_in_dim(ids_flat, offs[c], s).reshape(
                NW, s // (NW * GR), GR
            ),
            emb_weight,
        )
        for c, s in enumerate(chunk_sizes)
    ]
    y = None
    for c, s in enumerate(chunk_sizes):
        y = _tc_project_chunk(
            y, gathered[c], w_bf, pos_tiled, n, offs[c] // block_m, block_m
        )
    return y.reshape(b, l, D_OUT)
